# baseline (speedup 1.0000x reference)
import jax
import jax.numpy as jnp
from jax.experimental import pallas as pl
from jax.experimental.pallas import tpu as pltpu

P = 8
H = 512
B = 1024
EF = 512
E = EF + P
T = 128
BP = B + T
HALF_PI = 1.5707963267948966
HI = jax.lax.Precision.HIGHEST


def _iota(shape, dim):
    return jax.lax.broadcasted_iota(jnp.int32, shape, dim)


def _encode(obs, freq):
    sel = (_iota((2 * P, EF), 0) == _iota((2 * P, EF), 1) // 64).astype(jnp.float32)
    a = jnp.dot(obs, sel, precision=HI, preferred_element_type=jnp.float32)
    tm = (_iota((32, EF), 1) % 32 == _iota((32, EF), 0)).astype(jnp.float32)
    freq_r = jnp.dot(freq, tm, precision=HI, preferred_element_type=jnp.float32)
    cosmask = (_iota((1, EF), 1) // 32) % 2 == 0
    off = jnp.where(cosmask, HALF_PI, 0.0)
    enc512 = jnp.sin(a * freq_r + off)
    return jnp.concatenate([enc512, obs[:, P:2 * P]], axis=1)


def _route_kernel(obs_ref, freq_ref, enc_ref, starts_ref, pos_ref):
    obs = obs_ref[...]
    one_hot = obs[:, P:2 * P]
    col8 = _iota((B, P), 1)
    mx = jnp.max(one_hot, axis=1, keepdims=True)
    idx = jnp.where(one_hot == mx, col8, P)
    pid = jnp.min(idx, axis=1, keepdims=True)
    mt = (pid == col8).astype(jnp.float32)

    tri = (_iota((B, B), 1) <= _iota((B, B), 0)).astype(jnp.float32)
    csum = jnp.dot(tri, mt, preferred_element_type=jnp.float32)
    counts = csum[B - 1:B, :]
    stm = (_iota((P, 16), 0) < _iota((P, 16), 1)).astype(jnp.float32)
    starts16 = jnp.dot(counts, stm, precision=HI,
                       preferred_element_type=jnp.float32)
    starts_ref[...] = starts16.astype(jnp.int32)

    pos = jnp.sum(mt * (csum - 1.0 + starts16[:, :P]), axis=1, keepdims=True)
    pos_ref[...] = pos

    pit = (pos.astype(jnp.int32) == _iota((B, B), 1)).astype(jnp.float32)
    obs_sorted = jax.lax.dot_general(
        pit, obs, (((0,), (0,)), ((), ())), precision=HI,
        preferred_element_type=jnp.float32)
    enc_ref[pl.ds(0, B), :] = _encode(obs_sorted, freq_ref[...])


def _expert_kernel(starts_ref, enc_ref, pos_ref,
                   w1, b1, w2, b2, w3, b3, wd, bd,
                   u1, c1, u2, c2, u3, c3, ud, cd,
                   pi_ref, vf_ref, ys_s):
    g = pl.program_id(0)

    @pl.when(g == 0)
    def _init():
        ys_s[...] = jnp.zeros_like(ys_s)

    @pl.when(g <= P - 1)
    def _expert():
        s = starts_ref[g]
        e = starts_ref[g + 1]
        sa = (s // 8) * 8
        ntiles = (e - sa + T - 1) // T

        def body(t, carry):
            lo = sa + t * T
            x = enc_ref[pl.ds(lo, T), :]

            def mlp(w1r, b1r, w2r, b2r, w3r, b3r, wdr, bdr):
                h = jnp.maximum(
                    jnp.dot(x, w1r[0], preferred_element_type=jnp.float32) + b1r[0], 0.0)
                h = jnp.maximum(
                    jnp.dot(h, w2r[0], preferred_element_type=jnp.float32) + b2r[0], 0.0)
                h = jnp.maximum(
                    jnp.dot(h, w3r[0], preferred_element_type=jnp.float32) + b3r[0], 0.0)
                o = jnp.dot(h, wdr[0], preferred_element_type=jnp.float32) + bdr[0]
                return jnp.pi * jnp.tanh(o)

            ypi = mlp(w1, b1, w2, b2, w3, b3, wd, bd)
            yvf = mlp(u1, c1, u2, c2, u3, c3, ud, cd)
            y = jnp.concatenate([ypi, yvf], axis=1)

            rows = lo + _iota((T, 2), 0)
            valid = (rows >= s) & (rows < e)
            old = ys_s[pl.ds(lo, T), :]
            ys_s[pl.ds(lo, T), :] = jnp.where(valid, y, old)
            return carry

        jax.lax.fori_loop(0, ntiles, body, 0)

    @pl.when(g == P)
    def _unsort():
        pos = pos_ref[...].astype(jnp.int32)
        oh = (pos == _iota((B, B), 1)).astype(jnp.float32)
        y = jnp.dot(oh, ys_s[pl.ds(0, B), :], precision=HI,
                    preferred_element_type=jnp.float32)
        pi_ref[...] = y[:, 0:1]
        vf_ref[...] = y[:, 1:2]


def kernel(observations, frequencies, pW1, pb1, pW2, pb2, pW3, pb3, pWd, pbd,
           vW1, vb1, vW2, vb2, vW3, vb3, vWd, vbd):
    freq = frequencies.reshape(1, 32)

    enc_sorted, starts, pos = pl.pallas_call(
        _route_kernel,
        out_shape=[
            jax.ShapeDtypeStruct((BP, E), jnp.float32),
            jax.ShapeDtypeStruct((1, 16), jnp.int32),
            jax.ShapeDtypeStruct((B, 1), jnp.float32),
        ],
    )(observations, freq)

    def wsp(a):
        return pl.BlockSpec(
            (1,) + a.shape[1:],
            lambda g: (jnp.clip(g, 0, P - 1),) + (0,) * (a.ndim - 1))

    weight_args = (pW1, pb1[:, None, :], pW2, pb2[:, None, :],
                   pW3, pb3[:, None, :], pWd, pbd[:, None, :],
                   vW1, vb1[:, None, :], vW2, vb2[:, None, :],
                   vW3, vb3[:, None, :], vWd, vbd[:, None, :])

    latent_pi, latent_vf = pl.pallas_call(
        _expert_kernel,
        grid=(P + 1,),
        in_specs=[
            pl.BlockSpec(memory_space=pltpu.SMEM),
            pl.BlockSpec((BP, E), lambda g: (0, 0)),
            pl.BlockSpec((B, 1), lambda g: (0, 0)),
        ] + [wsp(a) for a in weight_args],
        out_specs=[
            pl.BlockSpec((B, 1), lambda g: (0, 0)),
            pl.BlockSpec((B, 1), lambda g: (0, 0)),
        ],
        out_shape=[
            jax.ShapeDtypeStruct((B, 1), jnp.float32),
            jax.ShapeDtypeStruct((B, 1), jnp.float32),
        ],
        scratch_shapes=[
            pltpu.VMEM((BP, 2), jnp.float32),
        ],
    )(starts.reshape(16), enc_sorted, pos, *weight_args)

    return (latent_pi, latent_vf)


# speedup vs baseline: 1.0800x; 1.0800x over previous
import jax
import jax.numpy as jnp
from jax.experimental import pallas as pl
from jax.experimental.pallas import tpu as pltpu

P = 8
H = 512
B = 1024
EF = 512
E = EF + P
T = 128
BP = B + T
HALF_PI = 1.5707963267948966
HI = jax.lax.Precision.HIGHEST


def _iota(shape, dim):
    return jax.lax.broadcasted_iota(jnp.int32, shape, dim)


def _encode(obs, freq):
    sel = (_iota((2 * P, EF), 0) == _iota((2 * P, EF), 1) // 64).astype(jnp.float32)
    a = jnp.dot(obs, sel, precision=HI, preferred_element_type=jnp.float32)
    tm = (_iota((32, EF), 1) % 32 == _iota((32, EF), 0)).astype(jnp.float32)
    freq_r = jnp.dot(freq, tm, precision=HI, preferred_element_type=jnp.float32)
    cosmask = (_iota((1, EF), 1) // 32) % 2 == 0
    off = jnp.where(cosmask, HALF_PI, 0.0)
    enc512 = jnp.sin(a * freq_r + off)
    out = jnp.concatenate([enc512, obs[:, P:2 * P]], axis=1)
    return out.astype(jnp.bfloat16)


def _route_kernel(obs_ref, freq_ref, enc_ref, starts_ref, pos_ref):
    obs = obs_ref[...]
    one_hot = obs[:, P:2 * P]
    col8 = _iota((B, P), 1)
    mx = jnp.max(one_hot, axis=1, keepdims=True)
    idx = jnp.where(one_hot == mx, col8, P)
    pid = jnp.min(idx, axis=1, keepdims=True)
    mt = (pid == col8).astype(jnp.float32)

    tri = (_iota((B, B), 1) <= _iota((B, B), 0)).astype(jnp.float32)
    csum = jnp.dot(tri, mt, preferred_element_type=jnp.float32)
    counts = csum[B - 1:B, :]
    stm = (_iota((P, 16), 0) < _iota((P, 16), 1)).astype(jnp.float32)
    starts16 = jnp.dot(counts, stm, precision=HI,
                       preferred_element_type=jnp.float32)
    starts_ref[...] = starts16.astype(jnp.int32)

    pos = jnp.sum(mt * (csum - 1.0 + starts16[:, :P]), axis=1, keepdims=True)
    pos_ref[...] = pos

    pit = (pos.astype(jnp.int32) == _iota((B, B), 1)).astype(jnp.float32)
    obs_sorted = jax.lax.dot_general(
        pit, obs, (((0,), (0,)), ((), ())), precision=HI,
        preferred_element_type=jnp.float32)
    enc_ref[pl.ds(0, B), :] = _encode(obs_sorted, freq_ref[...])


def _expert_kernel(starts_ref, enc_ref, pos_ref,
                   w1, b1, w2, b2, w3, b3, wd, bd,
                   u1, c1, u2, c2, u3, c3, ud, cd,
                   pi_ref, vf_ref,
                   ys_s, w1b, w2b, w3b, u2b, u3b, hdb):
    g = pl.program_id(0)

    @pl.when(g == 0)
    def _init():
        ys_s[...] = jnp.zeros_like(ys_s)

    @pl.when(g <= P - 1)
    def _expert():
        bf = jnp.bfloat16
        w1b[:, :H] = w1[0].astype(bf)
        w1b[:, H:] = u1[0].astype(bf)
        w2b[...] = w2[0].astype(bf)
        w3b[...] = w3[0].astype(bf)
        u2b[...] = u2[0].astype(bf)
        u3b[...] = u3[0].astype(bf)
        hdb[:, 0:1] = wd[0].astype(bf)
        hdb[:, 1:2] = ud[0].astype(bf)

        s = starts_ref[g]
        e = starts_ref[g + 1]
        sa = (s // 8) * 8
        ntiles = (e - sa + T - 1) // T
        b1c = jnp.concatenate([b1[0], c1[0]], axis=1)

        def body(t, carry):
            lo = sa + t * T
            x = enc_ref[pl.ds(lo, T), :]
            h1 = jnp.dot(x, w1b[...], preferred_element_type=jnp.float32) + b1c
            h1 = jnp.maximum(h1, 0.0).astype(bf)

            def tail(h, w2r, b2r, w3r, b3r):
                h = jnp.dot(h, w2r[...], preferred_element_type=jnp.float32) + b2r[0]
                h = jnp.maximum(h, 0.0).astype(bf)
                h = jnp.dot(h, w3r[...], preferred_element_type=jnp.float32) + b3r[0]
                return jnp.maximum(h, 0.0).astype(bf)

            h3p = tail(h1[:, :H], w2b, b2, w3b, b3)
            h3v = tail(h1[:, H:], u2b, c2, u3b, c3)
            op = jnp.dot(h3p, hdb[:, 0:1], preferred_element_type=jnp.float32)
            ov = jnp.dot(h3v, hdb[:, 1:2], preferred_element_type=jnp.float32)
            o = jnp.concatenate([op + bd[0], ov + cd[0]], axis=1)
            y = jnp.pi * jnp.tanh(o)

            rows = lo + _iota((T, 2), 0)
            valid = (rows >= s) & (rows < e)
            old = ys_s[pl.ds(lo, T), :]
            ys_s[pl.ds(lo, T), :] = jnp.where(valid, y, old)
            return carry

        jax.lax.fori_loop(0, ntiles, body, 0)

    @pl.when(g == P)
    def _unsort():
        pos = pos_ref[...].astype(jnp.int32)
        oh = (pos == _iota((B, B), 1)).astype(jnp.float32)
        y = jnp.dot(oh, ys_s[pl.ds(0, B), :],
                    preferred_element_type=jnp.float32)
        pi_ref[...] = y[:, 0:1]
        vf_ref[...] = y[:, 1:2]


def kernel(observations, frequencies, pW1, pb1, pW2, pb2, pW3, pb3, pWd, pbd,
           vW1, vb1, vW2, vb2, vW3, vb3, vWd, vbd):
    freq = frequencies.reshape(1, 32)

    enc_sorted, starts, pos = pl.pallas_call(
        _route_kernel,
        out_shape=[
            jax.ShapeDtypeStruct((BP, E), jnp.bfloat16),
            jax.ShapeDtypeStruct((1, 16), jnp.int32),
            jax.ShapeDtypeStruct((B, 1), jnp.float32),
        ],
    )(observations, freq)

    def wsp(a):
        return pl.BlockSpec(
            (1,) + a.shape[1:],
            lambda g: (jnp.clip(g, 0, P - 1),) + (0,) * (a.ndim - 1))

    weight_args = (pW1, pb1[:, None, :], pW2, pb2[:, None, :],
                   pW3, pb3[:, None, :], pWd, pbd[:, None, :],
                   vW1, vb1[:, None, :], vW2, vb2[:, None, :],
                   vW3, vb3[:, None, :], vWd, vbd[:, None, :])

    latent_pi, latent_vf = pl.pallas_call(
        _expert_kernel,
        grid=(P + 1,),
        in_specs=[
            pl.BlockSpec(memory_space=pltpu.SMEM),
            pl.BlockSpec((BP, E), lambda g: (0, 0)),
            pl.BlockSpec((B, 1), lambda g: (0, 0)),
        ] + [wsp(a) for a in weight_args],
        out_specs=[
            pl.BlockSpec((B, 1), lambda g: (0, 0)),
            pl.BlockSpec((B, 1), lambda g: (0, 0)),
        ],
        out_shape=[
            jax.ShapeDtypeStruct((B, 1), jnp.float32),
            jax.ShapeDtypeStruct((B, 1), jnp.float32),
        ],
        scratch_shapes=[
            pltpu.VMEM((BP, 2), jnp.float32),
            pltpu.VMEM((E, 2 * H), jnp.bfloat16),
            pltpu.VMEM((H, H), jnp.bfloat16),
            pltpu.VMEM((H, H), jnp.bfloat16),
            pltpu.VMEM((H, H), jnp.bfloat16),
            pltpu.VMEM((H, H), jnp.bfloat16),
            pltpu.VMEM((H, 2), jnp.bfloat16),
        ],
    )(starts.reshape(16), enc_sorted, pos, *weight_args)

    return (latent_pi, latent_vf)


# speedup vs baseline: 1.1575x; 1.0718x over previous
import jax
import jax.numpy as jnp
from jax.experimental import pallas as pl
from jax.experimental.pallas import tpu as pltpu

P = 8
H = 512
B = 1024
EF = 512
E = EF + P
T = 128
BP = B + T
HALF_PI = 1.5707963267948966
HI = jax.lax.Precision.HIGHEST


def _iota(shape, dim):
    return jax.lax.broadcasted_iota(jnp.int32, shape, dim)


def _encode(obs, freq):
    sel = (_iota((2 * P, EF), 0) == _iota((2 * P, EF), 1) // 64).astype(jnp.float32)
    a = jnp.dot(obs, sel, preferred_element_type=jnp.float32)
    tm = (_iota((32, EF), 1) % 32 == _iota((32, EF), 0)).astype(jnp.float32)
    freq_r = jnp.dot(freq, tm, preferred_element_type=jnp.float32)
    cosmask = (_iota((1, EF), 1) // 32) % 2 == 0
    off = jnp.where(cosmask, HALF_PI, 0.0)
    enc512 = jnp.sin(a * freq_r + off)
    out = jnp.concatenate([enc512, obs[:, P:2 * P]], axis=1)
    return out.astype(jnp.bfloat16)


def _route_kernel(obs_ref, freq_ref, enc_ref, starts_ref, pos_ref):
    obs = obs_ref[...]
    one_hot = obs[:, P:2 * P]
    col8 = _iota((B, P), 1)
    mx = jnp.max(one_hot, axis=1, keepdims=True)
    idx = jnp.where(one_hot == mx, col8, P)
    pid = jnp.min(idx, axis=1, keepdims=True)
    mt = (pid == col8).astype(jnp.float32)

    tri = (_iota((B, B), 1) <= _iota((B, B), 0)).astype(jnp.float32)
    csum = jnp.dot(tri, mt, preferred_element_type=jnp.float32)
    counts = csum[B - 1:B, :]
    stm = (_iota((P, 16), 0) < _iota((P, 16), 1)).astype(jnp.float32)
    starts16 = jnp.dot(counts, stm,
                       preferred_element_type=jnp.float32)
    starts_ref[...] = starts16.astype(jnp.int32)

    pos = jnp.sum(mt * (csum - 1.0 + starts16[:, :P]), axis=1, keepdims=True)
    pos_ref[...] = pos

    pit = (pos.astype(jnp.int32) == _iota((B, B), 1)).astype(jnp.float32)
    obs_sorted = jax.lax.dot_general(
        pit, obs, (((0,), (0,)), ((), ())),
        preferred_element_type=jnp.float32)
    enc_ref[pl.ds(0, B), :] = _encode(obs_sorted, freq_ref[...])


def _expert_kernel(starts_ref, enc_ref, pos_ref,
                   w1, b1, w2, b2, w3, b3, wd, bd,
                   u1, c1, u2, c2, u3, c3, ud, cd,
                   pi_ref, vf_ref,
                   ys_s, w1b, w2b, w3b, u2b, u3b, hdb):
    g = pl.program_id(0)

    @pl.when(g == 0)
    def _init():
        ys_s[...] = jnp.zeros_like(ys_s)

    @pl.when(g <= P - 1)
    def _expert():
        bf = jnp.bfloat16
        w1b[:, :H] = w1[0].astype(bf)
        w1b[:, H:] = u1[0].astype(bf)
        w2b[...] = w2[0].astype(bf)
        w3b[...] = w3[0].astype(bf)
        u2b[...] = u2[0].astype(bf)
        u3b[...] = u3[0].astype(bf)
        hdb[:, 0:1] = wd[0].astype(bf)
        hdb[:, 1:2] = ud[0].astype(bf)

        s = starts_ref[g]
        e = starts_ref[g + 1]
        sa = (s // 8) * 8
        ntiles = (e - sa + T - 1) // T
        b1c = jnp.concatenate([b1[0], c1[0]], axis=1)

        def body(t, carry):
            lo = sa + t * T
            x = enc_ref[pl.ds(lo, T), :]
            h1 = jnp.dot(x, w1b[...], preferred_element_type=jnp.float32) + b1c
            h1 = jnp.maximum(h1, 0.0).astype(bf)

            def tail(h, w2r, b2r, w3r, b3r):
                h = jnp.dot(h, w2r[...], preferred_element_type=jnp.float32) + b2r[0]
                h = jnp.maximum(h, 0.0).astype(bf)
                h = jnp.dot(h, w3r[...], preferred_element_type=jnp.float32) + b3r[0]
                return jnp.maximum(h, 0.0).astype(bf)

            h3p = tail(h1[:, :H], w2b, b2, w3b, b3)
            h3v = tail(h1[:, H:], u2b, c2, u3b, c3)
            op = jnp.dot(h3p, hdb[:, 0:1], preferred_element_type=jnp.float32)
            ov = jnp.dot(h3v, hdb[:, 1:2], preferred_element_type=jnp.float32)
            o = jnp.concatenate([op + bd[0], ov + cd[0]], axis=1)
            y = jnp.pi * jnp.tanh(o)

            rows = lo + _iota((T, 2), 0)
            valid = (rows >= s) & (rows < e)
            old = ys_s[pl.ds(lo, T), :]
            ys_s[pl.ds(lo, T), :] = jnp.where(valid, y, old)
            return carry

        jax.lax.fori_loop(0, ntiles, body, 0)

    @pl.when(g == P)
    def _unsort():
        pos = pos_ref[...].astype(jnp.int32)
        oh = (pos == _iota((B, B), 1)).astype(jnp.float32)
        y = jnp.dot(oh, ys_s[pl.ds(0, B), :],
                    preferred_element_type=jnp.float32)
        pi_ref[...] = y[:, 0:1]
        vf_ref[...] = y[:, 1:2]


def kernel(observations, frequencies, pW1, pb1, pW2, pb2, pW3, pb3, pWd, pbd,
           vW1, vb1, vW2, vb2, vW3, vb3, vWd, vbd):
    freq = frequencies.reshape(1, 32)

    enc_sorted, starts, pos = pl.pallas_call(
        _route_kernel,
        out_shape=[
            jax.ShapeDtypeStruct((BP, E), jnp.bfloat16),
            jax.ShapeDtypeStruct((1, 16), jnp.int32),
            jax.ShapeDtypeStruct((B, 1), jnp.float32),
        ],
    )(observations, freq)

    def wsp(a):
        return pl.BlockSpec(
            (1,) + a.shape[1:],
            lambda g: (jnp.clip(g, 0, P - 1),) + (0,) * (a.ndim - 1))

    weight_args = (pW1, pb1[:, None, :], pW2, pb2[:, None, :],
                   pW3, pb3[:, None, :], pWd, pbd[:, None, :],
                   vW1, vb1[:, None, :], vW2, vb2[:, None, :],
                   vW3, vb3[:, None, :], vWd, vbd[:, None, :])

    latent_pi, latent_vf = pl.pallas_call(
        _expert_kernel,
        grid=(P + 1,),
        in_specs=[
            pl.BlockSpec(memory_space=pltpu.SMEM),
            pl.BlockSpec((BP, E), lambda g: (0, 0)),
            pl.BlockSpec((B, 1), lambda g: (0, 0)),
        ] + [wsp(a) for a in weight_args],
        out_specs=[
            pl.BlockSpec((B, 1), lambda g: (0, 0)),
            pl.BlockSpec((B, 1), lambda g: (0, 0)),
        ],
        out_shape=[
            jax.ShapeDtypeStruct((B, 1), jnp.float32),
            jax.ShapeDtypeStruct((B, 1), jnp.float32),
        ],
        scratch_shapes=[
            pltpu.VMEM((BP, 2), jnp.float32),
            pltpu.VMEM((E, 2 * H), jnp.bfloat16),
            pltpu.VMEM((H, H), jnp.bfloat16),
            pltpu.VMEM((H, H), jnp.bfloat16),
            pltpu.VMEM((H, H), jnp.bfloat16),
            pltpu.VMEM((H, H), jnp.bfloat16),
            pltpu.VMEM((H, 2), jnp.bfloat16),
        ],
    )(starts.reshape(16), enc_sorted, pos, *weight_args)

    return (latent_pi, latent_vf)


# speedup vs baseline: 1.1800x; 1.0194x over previous
import jax
import jax.numpy as jnp
from jax.experimental import pallas as pl
from jax.experimental.pallas import tpu as pltpu

P = 8
H = 512
B = 1024
EF = 512
E = EF + P
T = 256
BP = B + T
HALF_PI = 1.5707963267948966
HI = jax.lax.Precision.HIGHEST


def _iota(shape, dim):
    return jax.lax.broadcasted_iota(jnp.int32, shape, dim)


def _encode(obs, freq):
    sel = (_iota((2 * P, EF), 0) == _iota((2 * P, EF), 1) // 64).astype(jnp.float32)
    a = jnp.dot(obs, sel, preferred_element_type=jnp.float32)
    tm = (_iota((32, EF), 1) % 32 == _iota((32, EF), 0)).astype(jnp.float32)
    freq_r = jnp.dot(freq, tm, preferred_element_type=jnp.float32)
    cosmask = (_iota((1, EF), 1) // 32) % 2 == 0
    off = jnp.where(cosmask, HALF_PI, 0.0)
    enc512 = jnp.sin(a * freq_r + off)
    out = jnp.concatenate([enc512, obs[:, P:2 * P]], axis=1)
    return out.astype(jnp.bfloat16)


def _route_kernel(obs_ref, freq_ref, enc_ref, starts_ref, pos_ref):
    obs = obs_ref[...]
    one_hot = obs[:, P:2 * P]
    col8 = _iota((B, P), 1)
    mx = jnp.max(one_hot, axis=1, keepdims=True)
    idx = jnp.where(one_hot == mx, col8, P)
    pid = jnp.min(idx, axis=1, keepdims=True)
    mt = (pid == col8).astype(jnp.float32)

    tri = (_iota((B, B), 1) <= _iota((B, B), 0)).astype(jnp.float32)
    csum = jnp.dot(tri, mt, preferred_element_type=jnp.float32)
    counts = csum[B - 1:B, :]
    stm = (_iota((P, 16), 0) < _iota((P, 16), 1)).astype(jnp.float32)
    starts16 = jnp.dot(counts, stm,
                       preferred_element_type=jnp.float32)
    starts_ref[...] = starts16.astype(jnp.int32)

    pos = jnp.sum(mt * (csum - 1.0 + starts16[:, :P]), axis=1, keepdims=True)
    pos_ref[...] = pos

    pit = (pos.astype(jnp.int32) == _iota((B, B), 1)).astype(jnp.float32)
    obs_sorted = jax.lax.dot_general(
        pit, obs, (((0,), (0,)), ((), ())),
        preferred_element_type=jnp.float32)
    enc_ref[pl.ds(0, B), :] = _encode(obs_sorted, freq_ref[...])


def _expert_kernel(starts_ref, enc_ref, pos_ref,
                   w1, b1, w2, b2, w3, b3, wd, bd,
                   u1, c1, u2, c2, u3, c3, ud, cd,
                   pi_ref, vf_ref,
                   ys_s, w1b, w2b, w3b, u2b, u3b, hdb):
    g = pl.program_id(0)

    @pl.when(g == 0)
    def _init():
        ys_s[...] = jnp.zeros_like(ys_s)

    @pl.when(g <= P - 1)
    def _expert():
        bf = jnp.bfloat16
        w1b[:, :H] = w1[0].astype(bf)
        w1b[:, H:] = u1[0].astype(bf)
        w2b[...] = w2[0].astype(bf)
        w3b[...] = w3[0].astype(bf)
        u2b[...] = u2[0].astype(bf)
        u3b[...] = u3[0].astype(bf)
        hdb[:, 0:1] = wd[0].astype(bf)
        hdb[:, 1:2] = ud[0].astype(bf)

        s = starts_ref[g]
        e = starts_ref[g + 1]
        sa = (s // 8) * 8
        ntiles = (e - sa + T - 1) // T
        b1c = jnp.concatenate([b1[0], c1[0]], axis=1)

        def body(t, carry):
            lo = sa + t * T
            x = enc_ref[pl.ds(lo, T), :]
            h1 = jnp.dot(x, w1b[...], preferred_element_type=jnp.float32) + b1c
            h1 = jnp.maximum(h1, 0.0).astype(bf)

            def tail(h, w2r, b2r, w3r, b3r):
                h = jnp.dot(h, w2r[...], preferred_element_type=jnp.float32) + b2r[0]
                h = jnp.maximum(h, 0.0).astype(bf)
                h = jnp.dot(h, w3r[...], preferred_element_type=jnp.float32) + b3r[0]
                return jnp.maximum(h, 0.0).astype(bf)

            h3p = tail(h1[:, :H], w2b, b2, w3b, b3)
            h3v = tail(h1[:, H:], u2b, c2, u3b, c3)
            op = jnp.dot(h3p, hdb[:, 0:1], preferred_element_type=jnp.float32)
            ov = jnp.dot(h3v, hdb[:, 1:2], preferred_element_type=jnp.float32)
            o = jnp.concatenate([op + bd[0], ov + cd[0]], axis=1)
            y = jnp.pi * jnp.tanh(o)

            rows = lo + _iota((T, 2), 0)
            valid = (rows >= s) & (rows < e)
            old = ys_s[pl.ds(lo, T), :]
            ys_s[pl.ds(lo, T), :] = jnp.where(valid, y, old)
            return carry

        jax.lax.fori_loop(0, ntiles, body, 0)

    @pl.when(g == P)
    def _unsort():
        pos = pos_ref[...].astype(jnp.int32)
        oh = (pos == _iota((B, B), 1)).astype(jnp.float32)
        y = jnp.dot(oh, ys_s[pl.ds(0, B), :],
                    preferred_element_type=jnp.float32)
        pi_ref[...] = y[:, 0:1]
        vf_ref[...] = y[:, 1:2]


def kernel(observations, frequencies, pW1, pb1, pW2, pb2, pW3, pb3, pWd, pbd,
           vW1, vb1, vW2, vb2, vW3, vb3, vWd, vbd):
    freq = frequencies.reshape(1, 32)

    enc_sorted, starts, pos = pl.pallas_call(
        _route_kernel,
        out_shape=[
            jax.ShapeDtypeStruct((BP, E), jnp.bfloat16),
            jax.ShapeDtypeStruct((1, 16), jnp.int32),
            jax.ShapeDtypeStruct((B, 1), jnp.float32),
        ],
    )(observations, freq)

    def wsp(a):
        return pl.BlockSpec(
            (1,) + a.shape[1:],
            lambda g: (jnp.clip(g, 0, P - 1),) + (0,) * (a.ndim - 1))

    weight_args = (pW1, pb1[:, None, :], pW2, pb2[:, None, :],
                   pW3, pb3[:, None, :], pWd, pbd[:, None, :],
                   vW1, vb1[:, None, :], vW2, vb2[:, None, :],
                   vW3, vb3[:, None, :], vWd, vbd[:, None, :])

    latent_pi, latent_vf = pl.pallas_call(
        _expert_kernel,
        grid=(P + 1,),
        in_specs=[
            pl.BlockSpec(memory_space=pltpu.SMEM),
            pl.BlockSpec((BP, E), lambda g: (0, 0)),
            pl.BlockSpec((B, 1), lambda g: (0, 0)),
        ] + [wsp(a) for a in weight_args],
        out_specs=[
            pl.BlockSpec((B, 1), lambda g: (0, 0)),
            pl.BlockSpec((B, 1), lambda g: (0, 0)),
        ],
        out_shape=[
            jax.ShapeDtypeStruct((B, 1), jnp.float32),
            jax.ShapeDtypeStruct((B, 1), jnp.float32),
        ],
        scratch_shapes=[
            pltpu.VMEM((BP, 2), jnp.float32),
            pltpu.VMEM((E, 2 * H), jnp.bfloat16),
            pltpu.VMEM((H, H), jnp.bfloat16),
            pltpu.VMEM((H, H), jnp.bfloat16),
            pltpu.VMEM((H, H), jnp.bfloat16),
            pltpu.VMEM((H, H), jnp.bfloat16),
            pltpu.VMEM((H, 2), jnp.bfloat16),
        ],
    )(starts.reshape(16), enc_sorted, pos, *weight_args)

    return (latent_pi, latent_vf)
